# baseline (speedup 1.0000x reference)
import functools

import jax
import jax.numpy as jnp
from jax import lax
from jax.experimental import pallas as pl
from jax.experimental.pallas import tpu as pltpu
from jax.experimental.pallas import tpu_sc as plsc

_NUM_ENT = 1000000
_D = 64
_B = 16384
_EPS = 1e-5
_NREL = 5

_NC, _NS = 2, 16
_NW = _NC * _NS
_ROWS = 2 * _B
_WINW = 256
_NFULLW = 3906
_TAIL = _NFULLW * _WINW
_WPW = _NFULLW // _NW
_OUTR = _ROWS + 16


def _win_base(w):
    return (_WPW * w + jnp.minimum(w, 2)) * _WINW


def _sc_gather(table, idx):
    mesh = plsc.VectorSubcoreMesh(core_axis_name="c", subcore_axis_name="s")

    @functools.partial(
        pl.kernel,
        out_type=jax.ShapeDtypeStruct((_OUTR, 128), jnp.float32),
        mesh=mesh,
        scratch_types=[
            pltpu.VMEM((1024,), jnp.int32),
            pltpu.VMEM((_ROWS,), jnp.int32),
            pltpu.VMEM((_ROWS,), jnp.int32),
            pltpu.VMEM((3, _D, _WINW), jnp.float32),
            pltpu.VMEM((2, 16, 128), jnp.float32),
            pltpu.VMEM((2, 16), jnp.int32),
            pltpu.SMEM((18,), jnp.int32),
            pltpu.SemaphoreType.DMA,
            pltpu.SemaphoreType.DMA,
            pltpu.SemaphoreType.DMA,
        ],
        compiler_params=pltpu.CompilerParams(
            use_tc_tiling_on_sc=True, needs_layout_passes=False),
    )
    def k(tableT, tailT, idx_hbm, out_hbm, seg_v, list_v, match_v, win_v,
          stage_v, posb_v, bins_s, isem, wsem, osem):
        wid = lax.axis_index("s") * _NC + lax.axis_index("c")
        lo = _win_base(wid)
        nwin = _WPW + jnp.where(wid < 2, 1, 0)
        is31 = wid == _NW - 1
        nwin_t = nwin + jnp.where(is31, 1, 0)
        hi = jnp.where(is31, _NUM_ENT, lo + nwin * _WINW)

        def seg_body(sg, cnt):
            pltpu.sync_copy(idx_hbm.at[pl.ds(sg * 1024, 1024)], seg_v)

            def grp_body(g, cnt):
                ids = seg_v[pl.ds(g * 16, 16)]
                pos = lax.iota(jnp.int32, 16) + (sg * 1024 + g * 16)
                m = jnp.logical_and(ids >= lo, ids < hi)
                packed = ((ids - lo) << 16) | pos
                plsc.store_compressed(list_v.at[pl.ds(cnt, 16)], packed, mask=m)
                n = plsc.all_reduce_population_count(m)
                return cnt + n[0]

            return lax.fori_loop(0, 64, grp_body, cnt)

        cnt = lax.fori_loop(0, 32, seg_body, jnp.int32(0))
        ngrp = (cnt + 15) >> 4

        off = jnp.int32(0)
        for bb in range(16):
            bins_s[bb] = off

            def bin_body(q, off, bb=bb):
                packed = list_v[pl.ds(q * 16, 16)]
                valid = q * 16 + lax.iota(jnp.int32, 16) < cnt
                m = jnp.logical_and(valid, (packed >> 27) == bb)

                @pl.when(jnp.any(m))
                def _():
                    plsc.store_compressed(
                        match_v.at[pl.ds(off, 16)], packed, mask=m)

                n = plsc.all_reduce_population_count(m)
                return off + n[0]

            off = lax.fori_loop(0, ngrp, bin_body, off)
        bins_s[16] = off

        def fire(g):
            s = jax.lax.rem(g, 3)
            tail = jnp.logical_and(is31, g == nwin_t - 1)

            @pl.when(tail)
            def _():
                pltpu.async_copy(
                    tailT, win_v.at[s].at[:, pl.ds(0, 128)], wsem)

            @pl.when(jnp.logical_not(tail))
            def _():
                pltpu.async_copy(
                    tableT.at[:, pl.ds(lo + g * _WINW, _WINW)],
                    win_v.at[s], wsem)

        def wait_win(g):
            s = jax.lax.rem(g, 3)
            tail = jnp.logical_and(is31, g == nwin_t - 1)

            @pl.when(tail)
            def _():
                pltpu.make_async_copy(
                    tailT, win_v.at[s].at[:, pl.ds(0, 128)], wsem).wait()

            @pl.when(jnp.logical_not(tail))
            def _():
                pltpu.make_async_copy(
                    tableT.at[:, pl.ds(lo + g * _WINW, _WINW)],
                    win_v.at[s], wsem).wait()

        def wait_scat(s):
            pltpu.make_async_copy(
                stage_v.at[s], out_hbm.at[posb_v.at[s]], osem).wait()

        fire(jnp.int32(0))
        fire(jnp.int32(1))
        fire(jnp.int32(2))

        def win_body(g, sct):
            s = jax.lax.rem(g, 3)
            wait_win(g)
            wbase = g * _WINW

            qlo = bins_s[g >> 3]
            qhi = bins_s[(g >> 3) + 1]
            q0 = qlo >> 4

            def scan_body(q, nm):
                base16 = (q0 + q) * 16
                packed = match_v[pl.ds(base16, 16)]
                rel = packed >> 16
                lanei = base16 + lax.iota(jnp.int32, 16)
                valid = jnp.logical_and(lanei >= qlo, lanei < qhi)
                m = jnp.logical_and(valid, jnp.logical_and(
                    rel >= wbase, rel < wbase + _WINW))

                @pl.when(jnp.any(m))
                def _():
                    plsc.store_compressed(
                        list_v.at[pl.ds(nm, 16)], packed, mask=m)

                n = plsc.all_reduce_population_count(m)
                return nm + n[0]

            nm = lax.fori_loop(0, ((qhi + 15) >> 4) - q0, scan_body,
                               jnp.int32(0))

            def ext_body(e, sct):
                ss = sct & 1

                @pl.when(sct >= 2)
                def _():
                    wait_scat(ss)

                packed = list_v[pl.ds(e * 16, 16)]
                lanei = lax.iota(jnp.int32, 16)
                valid = e * 16 + lanei < nm
                rel_in = (packed >> 16) - wbase
                rel_in = jnp.where(valid, rel_in, 0)
                posr = jnp.where(valid, packed & 0xFFFF, _ROWS + lanei)
                st = stage_v.at[ss]
                for j in range(_D):
                    vals = plsc.load_gather(
                        win_v.at[s], [jnp.full((16,), j, jnp.int32), rel_in])
                    plsc.store_scatter(
                        st, [lanei, jnp.full((16,), j, jnp.int32)], vals)
                posb_v[ss, :] = posr
                pltpu.async_copy(st, out_hbm.at[posb_v.at[ss]], osem)
                return sct + 1

            sct = lax.fori_loop(0, (nm + 15) >> 4, ext_body, sct)

            @pl.when(g + 3 < nwin_t)
            def _():
                fire(g + 3)

            return sct

        sct = lax.fori_loop(0, nwin_t, win_body, jnp.int32(0))

        @pl.when(sct >= 2)
        def _():
            wait_scat(sct & 1)

        @pl.when(sct >= 1)
        def _():
            wait_scat((sct - 1) & 1)

    tail_pad = jnp.pad(table[_TAIL:].T, ((0, 0), (0, 128 - (_NUM_ENT - _TAIL))))
    return k(table.T, tail_pad, idx)


_BB = 2048
_NB = _B // _BB


def _tc_body(h_ref, t_ref, rels_ref, m_ref, g_ref, b_ref, w_ref,
             loss_ref, preds_ref, stats_ref, acc_ref):
    ph = pl.program_id(0)
    b = pl.program_id(1)

    Hb = h_ref[...][:, :_D]
    Tb = t_ref[...][:, :_D]

    @pl.when(jnp.logical_and(ph == 0, b == 0))
    def _init():
        stats_ref[...] = jnp.zeros_like(stats_ref)

    @pl.when(ph == 0)
    def _stats():
        stats_ref[0:1, :] += jnp.sum(Hb, axis=0, keepdims=True)
        stats_ref[1:2, :] += jnp.sum(Hb * Hb, axis=0, keepdims=True)
        stats_ref[2:3, :] += jnp.sum(Tb, axis=0, keepdims=True)
        stats_ref[3:4, :] += jnp.sum(Tb * Tb, axis=0, keepdims=True)

    @pl.when(ph == 1)
    def _decode():
        gamma = g_ref[...]
        beta = b_ref[...]
        mH = stats_ref[0:1, :] / _B
        vH = stats_ref[1:2, :] / _B - mH * mH
        mT = stats_ref[2:3, :] / _B
        vT = stats_ref[3:4, :] / _B - mT * mT
        Hn = (Hb - mH) / jnp.sqrt(vH + _EPS) * gamma + beta
        Tn = (Tb - mT) / jnp.sqrt(vT + _EPS) * gamma + beta

        u0 = jnp.dot(Hn, m_ref[0], preferred_element_type=jnp.float32)
        s0 = jnp.sum(u0 * Tn, axis=1)
        u1 = jnp.dot(Hn, m_ref[1], preferred_element_type=jnp.float32)
        s1 = jnp.sum(u1 * Tn, axis=1)

        logits = [s0 * w_ref[0, j] + s1 * w_ref[1, j] for j in range(_NREL)]
        m = logits[0]
        for j in range(1, _NREL):
            m = jnp.maximum(m, logits[j])
        es = [jnp.exp(lg - m) for lg in logits]
        se = es[0]
        for j in range(1, _NREL):
            se = se + es[j]
        lse = m + jnp.log(se)

        rels = rels_ref[...]
        pick = jnp.zeros_like(s0)
        wsum = jnp.zeros_like(s0)
        for j in range(_NREL):
            pick = pick + jnp.where(rels == j, logits[j], 0.0)
            wsum = wsum + (j + 1.0) * es[j]
        preds_ref[...] = wsum / se

        part = jnp.sum(lse - pick)

        @pl.when(b == 0)
        def _():
            acc_ref[0] = part

        @pl.when(b > 0)
        def _():
            acc_ref[0] += part

        @pl.when(b == _NB - 1)
        def _():
            loss_ref[0, 0] = acc_ref[0] / _B


def _tc_decode(rows, rels, rel_mats, gamma, beta, wscal):
    grid = (2, _NB)
    return pl.pallas_call(
        _tc_body,
        grid=grid,
        in_specs=[
            pl.BlockSpec((_BB, 128), lambda ph, b: (b, 0)),
            pl.BlockSpec((_BB, 128), lambda ph, b: (b + _NB, 0)),
            pl.BlockSpec((_BB,), lambda ph, b: (b,)),
            pl.BlockSpec((2, _D, _D), lambda ph, b: (0, 0, 0)),
            pl.BlockSpec((1, _D), lambda ph, b: (0, 0)),
            pl.BlockSpec((1, _D), lambda ph, b: (0, 0)),
            pl.BlockSpec(memory_space=pltpu.SMEM),
        ],
        out_specs=[
            pl.BlockSpec(memory_space=pltpu.SMEM),
            pl.BlockSpec((_BB,), lambda ph, b: (b,)),
        ],
        out_shape=[
            jax.ShapeDtypeStruct((1, 1), jnp.float32),
            jax.ShapeDtypeStruct((_B,), jnp.float32),
        ],
        scratch_shapes=[
            pltpu.VMEM((8, _D), jnp.float32),
            pltpu.SMEM((1,), jnp.float32),
        ],
    )(rows, rows, rels, rel_mats, gamma, beta, wscal)


def kernel(pos_edges, encoder_weight, bn_gamma, bn_beta, rel_embeds, weight_scalars):
    heads = pos_edges[:, 0]
    rels = pos_edges[:, 1]
    tails = pos_edges[:, 2]
    idx = jnp.concatenate([heads, tails])

    rows = _sc_gather(encoder_weight, idx)

    rel_mats = rel_embeds.reshape(2, _D, _D)
    gamma = bn_gamma.reshape(1, _D)
    beta = bn_beta.reshape(1, _D)

    loss_arr, preds = _tc_decode(rows, rels, rel_mats, gamma, beta, weight_scalars)
    return loss_arr[0, 0], preds.reshape(_B, 1)


# speedup vs baseline: 1.2223x; 1.2223x over previous
import functools

import jax
import jax.numpy as jnp
from jax import lax
from jax.experimental import pallas as pl
from jax.experimental.pallas import tpu as pltpu
from jax.experimental.pallas import tpu_sc as plsc

_NUM_ENT = 1000000
_D = 64
_B = 16384
_EPS = 1e-5
_NREL = 5

_NC, _NS = 2, 16
_NW = _NC * _NS
_ROWS = 2 * _B
_WINW = 384
_NFULLW = 2604
_TAIL = _NFULLW * _WINW
_WPW = _NFULLW // _NW
_NWREM = _NFULLW - _WPW * _NW
_OUTR = _ROWS + 16


def _win_base(w):
    return (_WPW * w + jnp.minimum(w, _NWREM)) * _WINW


def _sc_gather(table, idx):
    mesh = plsc.VectorSubcoreMesh(core_axis_name="c", subcore_axis_name="s")

    @functools.partial(
        pl.kernel,
        out_type=jax.ShapeDtypeStruct((_OUTR, 128), jnp.float32),
        mesh=mesh,
        scratch_types=[
            pltpu.VMEM((1024,), jnp.int32),
            pltpu.VMEM((_ROWS,), jnp.int32),
            pltpu.VMEM((_ROWS,), jnp.int32),
            pltpu.VMEM((2, _D, _WINW), jnp.float32),
            pltpu.VMEM((2, 16, 128), jnp.float32),
            pltpu.VMEM((2, 16), jnp.int32),
            pltpu.SMEM((18,), jnp.int32),
            pltpu.SemaphoreType.DMA,
            pltpu.SemaphoreType.DMA,
            pltpu.SemaphoreType.DMA,
        ],
        compiler_params=pltpu.CompilerParams(
            use_tc_tiling_on_sc=True, needs_layout_passes=False),
    )
    def k(tableT, tailT, idx_hbm, out_hbm, seg_v, list_v, match_v, win_v,
          stage_v, posb_v, bins_s, isem, wsem, osem):
        wid = lax.axis_index("s") * _NC + lax.axis_index("c")
        lo = _win_base(wid)
        nwin = _WPW + jnp.where(wid < _NWREM, 1, 0)
        is31 = wid == _NW - 1
        nwin_t = nwin + jnp.where(is31, 1, 0)
        hi = jnp.where(is31, _NUM_ENT, lo + nwin * _WINW)

        def seg_body(sg, cnt):
            pltpu.sync_copy(idx_hbm.at[pl.ds(sg * 1024, 1024)], seg_v)

            def grp_body(g, cnt):
                ids = seg_v[pl.ds(g * 16, 16)]
                pos = lax.iota(jnp.int32, 16) + (sg * 1024 + g * 16)
                m = jnp.logical_and(ids >= lo, ids < hi)
                packed = ((ids - lo) << 16) | pos
                plsc.store_compressed(list_v.at[pl.ds(cnt, 16)], packed, mask=m)
                n = plsc.all_reduce_population_count(m)
                return cnt + n[0]

            return lax.fori_loop(0, 64, grp_body, cnt)

        cnt = lax.fori_loop(0, 32, seg_body, jnp.int32(0))
        ngrp = (cnt + 15) >> 4

        def fire(g):
            s = g & 1
            tail = jnp.logical_and(is31, g == nwin_t - 1)

            @pl.when(tail)
            def _():
                pltpu.async_copy(
                    tailT, win_v.at[s].at[:, pl.ds(0, 128)], wsem)

            @pl.when(jnp.logical_not(tail))
            def _():
                pltpu.async_copy(
                    tableT.at[:, pl.ds(lo + g * _WINW, _WINW)],
                    win_v.at[s], wsem)

        def wait_win(g):
            s = g & 1
            tail = jnp.logical_and(is31, g == nwin_t - 1)

            @pl.when(tail)
            def _():
                pltpu.make_async_copy(
                    tailT, win_v.at[s].at[:, pl.ds(0, 128)], wsem).wait()

            @pl.when(jnp.logical_not(tail))
            def _():
                pltpu.make_async_copy(
                    tableT.at[:, pl.ds(lo + g * _WINW, _WINW)],
                    win_v.at[s], wsem).wait()

        def wait_scat(s):
            pltpu.make_async_copy(
                stage_v.at[s], out_hbm.at[posb_v.at[s]], osem).wait()

        fire(jnp.int32(0))
        fire(jnp.int32(1))

        def win_body(g, sct):
            s = g & 1
            wait_win(g)
            wbase = g * _WINW

            def scan_body(q, nm):
                packed = list_v[pl.ds(q * 16, 16)]
                rel = packed >> 16
                valid = q * 16 + lax.iota(jnp.int32, 16) < cnt
                m = jnp.logical_and(valid, jnp.logical_and(
                    rel >= wbase, rel < wbase + _WINW))

                @pl.when(jnp.any(m))
                def _():
                    plsc.store_compressed(
                        match_v.at[pl.ds(nm, 16)], packed, mask=m)

                n = plsc.all_reduce_population_count(m)
                return nm + n[0]

            nm = lax.fori_loop(0, ngrp, scan_body, jnp.int32(0))

            def ext_body(e, sct):
                ss = sct & 1

                @pl.when(sct >= 2)
                def _():
                    wait_scat(ss)

                packed = match_v[pl.ds(e * 16, 16)]
                lanei = lax.iota(jnp.int32, 16)
                valid = e * 16 + lanei < nm
                rel_in = (packed >> 16) - wbase
                rel_in = jnp.where(valid, rel_in, 0)
                posr = jnp.where(valid, packed & 0xFFFF, _ROWS + lanei)
                st = stage_v.at[ss]
                for j in range(_D):
                    vals = plsc.load_gather(
                        win_v.at[s], [jnp.full((16,), j, jnp.int32), rel_in])
                    plsc.store_scatter(
                        st, [lanei, jnp.full((16,), j, jnp.int32)], vals)
                posb_v[ss, :] = posr
                pltpu.async_copy(st, out_hbm.at[posb_v.at[ss]], osem)
                return sct + 1

            sct = lax.fori_loop(0, (nm + 15) >> 4, ext_body, sct)

            @pl.when(g + 2 < nwin_t)
            def _():
                fire(g + 2)

            return sct

        sct = lax.fori_loop(0, nwin_t, win_body, jnp.int32(0))

        @pl.when(sct >= 2)
        def _():
            wait_scat(sct & 1)

        @pl.when(sct >= 1)
        def _():
            wait_scat((sct - 1) & 1)

    tail_pad = jnp.pad(table[_TAIL:].T, ((0, 0), (0, 128 - (_NUM_ENT - _TAIL))))
    return k(table.T, tail_pad, idx)


_BB = 2048
_NB = _B // _BB


def _tc_body(h_ref, t_ref, rels_ref, m_ref, g_ref, b_ref, w_ref,
             loss_ref, preds_ref, stats_ref, acc_ref):
    ph = pl.program_id(0)
    b = pl.program_id(1)

    Hb = h_ref[...][:, :_D]
    Tb = t_ref[...][:, :_D]

    @pl.when(jnp.logical_and(ph == 0, b == 0))
    def _init():
        stats_ref[...] = jnp.zeros_like(stats_ref)

    @pl.when(ph == 0)
    def _stats():
        stats_ref[0:1, :] += jnp.sum(Hb, axis=0, keepdims=True)
        stats_ref[1:2, :] += jnp.sum(Hb * Hb, axis=0, keepdims=True)
        stats_ref[2:3, :] += jnp.sum(Tb, axis=0, keepdims=True)
        stats_ref[3:4, :] += jnp.sum(Tb * Tb, axis=0, keepdims=True)

    @pl.when(ph == 1)
    def _decode():
        gamma = g_ref[...]
        beta = b_ref[...]
        mH = stats_ref[0:1, :] / _B
        vH = stats_ref[1:2, :] / _B - mH * mH
        mT = stats_ref[2:3, :] / _B
        vT = stats_ref[3:4, :] / _B - mT * mT
        Hn = (Hb - mH) / jnp.sqrt(vH + _EPS) * gamma + beta
        Tn = (Tb - mT) / jnp.sqrt(vT + _EPS) * gamma + beta

        u0 = jnp.dot(Hn, m_ref[0], preferred_element_type=jnp.float32)
        s0 = jnp.sum(u0 * Tn, axis=1)
        u1 = jnp.dot(Hn, m_ref[1], preferred_element_type=jnp.float32)
        s1 = jnp.sum(u1 * Tn, axis=1)

        logits = [s0 * w_ref[0, j] + s1 * w_ref[1, j] for j in range(_NREL)]
        m = logits[0]
        for j in range(1, _NREL):
            m = jnp.maximum(m, logits[j])
        es = [jnp.exp(lg - m) for lg in logits]
        se = es[0]
        for j in range(1, _NREL):
            se = se + es[j]
        lse = m + jnp.log(se)

        rels = rels_ref[...]
        pick = jnp.zeros_like(s0)
        wsum = jnp.zeros_like(s0)
        for j in range(_NREL):
            pick = pick + jnp.where(rels == j, logits[j], 0.0)
            wsum = wsum + (j + 1.0) * es[j]
        preds_ref[...] = wsum / se

        part = jnp.sum(lse - pick)

        @pl.when(b == 0)
        def _():
            acc_ref[0] = part

        @pl.when(b > 0)
        def _():
            acc_ref[0] += part

        @pl.when(b == _NB - 1)
        def _():
            loss_ref[0, 0] = acc_ref[0] / _B


def _tc_decode(rows, rels, rel_mats, gamma, beta, wscal):
    grid = (2, _NB)
    return pl.pallas_call(
        _tc_body,
        grid=grid,
        in_specs=[
            pl.BlockSpec((_BB, 128), lambda ph, b: (b, 0)),
            pl.BlockSpec((_BB, 128), lambda ph, b: (b + _NB, 0)),
            pl.BlockSpec((_BB,), lambda ph, b: (b,)),
            pl.BlockSpec((2, _D, _D), lambda ph, b: (0, 0, 0)),
            pl.BlockSpec((1, _D), lambda ph, b: (0, 0)),
            pl.BlockSpec((1, _D), lambda ph, b: (0, 0)),
            pl.BlockSpec(memory_space=pltpu.SMEM),
        ],
        out_specs=[
            pl.BlockSpec(memory_space=pltpu.SMEM),
            pl.BlockSpec((_BB,), lambda ph, b: (b,)),
        ],
        out_shape=[
            jax.ShapeDtypeStruct((1, 1), jnp.float32),
            jax.ShapeDtypeStruct((_B,), jnp.float32),
        ],
        scratch_shapes=[
            pltpu.VMEM((8, _D), jnp.float32),
            pltpu.SMEM((1,), jnp.float32),
        ],
    )(rows, rows, rels, rel_mats, gamma, beta, wscal)


def kernel(pos_edges, encoder_weight, bn_gamma, bn_beta, rel_embeds, weight_scalars):
    heads = pos_edges[:, 0]
    rels = pos_edges[:, 1]
    tails = pos_edges[:, 2]
    idx = jnp.concatenate([heads, tails])

    rows = _sc_gather(encoder_weight, idx)

    rel_mats = rel_embeds.reshape(2, _D, _D)
    gamma = bn_gamma.reshape(1, _D)
    beta = bn_beta.reshape(1, _D)

    loss_arr, preds = _tc_decode(rows, rels, rel_mats, gamma, beta, weight_scalars)
    return loss_arr[0, 0], preds.reshape(_B, 1)


# speedup vs baseline: 1.2233x; 1.0007x over previous
import functools

import jax
import jax.numpy as jnp
from jax import lax
from jax.experimental import pallas as pl
from jax.experimental.pallas import tpu as pltpu
from jax.experimental.pallas import tpu_sc as plsc

_NUM_ENT = 1000000
_D = 64
_B = 16384
_EPS = 1e-5
_NREL = 5

_NC, _NS = 2, 16
_NW = _NC * _NS
_ROWS = 2 * _B
_WINW = 384
_NFULLW = 2604
_TAIL = _NFULLW * _WINW
_WPW = _NFULLW // _NW
_NWREM = _NFULLW - _WPW * _NW
_OUTR = _ROWS + 16


def _win_base(w):
    return (_WPW * w + jnp.minimum(w, _NWREM)) * _WINW


def _sc_gather(table, idx):
    mesh = plsc.VectorSubcoreMesh(core_axis_name="c", subcore_axis_name="s")

    @functools.partial(
        pl.kernel,
        out_type=jax.ShapeDtypeStruct((_OUTR, 128), jnp.float32),
        mesh=mesh,
        scratch_types=[
            pltpu.VMEM((1024,), jnp.int32),
            pltpu.VMEM((_ROWS,), jnp.int32),
            pltpu.VMEM((_ROWS,), jnp.int32),
            pltpu.VMEM((2, _D, _WINW), jnp.float32),
            pltpu.VMEM((2, 16, 128), jnp.float32),
            pltpu.VMEM((2, 16), jnp.int32),
            pltpu.SMEM((18,), jnp.int32),
            pltpu.SemaphoreType.DMA,
            pltpu.SemaphoreType.DMA,
            pltpu.SemaphoreType.DMA,
        ],
        compiler_params=pltpu.CompilerParams(
            use_tc_tiling_on_sc=True, needs_layout_passes=False),
    )
    def k(tableT, tailT, idx_hbm, out_hbm, seg_v, list_v, match_v, win_v,
          stage_v, posb_v, bins_s, isem, wsem, osem):
        wid = lax.axis_index("s") * _NC + lax.axis_index("c")
        lo = _win_base(wid)
        nwin = _WPW + jnp.where(wid < _NWREM, 1, 0)
        is31 = wid == _NW - 1
        nwin_t = nwin + jnp.where(is31, 1, 0)
        hi = jnp.where(is31, _NUM_ENT, lo + nwin * _WINW)

        def seg_body(sg, cnt):
            pltpu.sync_copy(idx_hbm.at[pl.ds(sg * 1024, 1024)], seg_v)

            def grp_body(g, cnt):
                ids = seg_v[pl.ds(g * 16, 16)]
                pos = lax.iota(jnp.int32, 16) + (sg * 1024 + g * 16)
                m = jnp.logical_and(ids >= lo, ids < hi)
                packed = ((ids - lo) << 16) | pos
                plsc.store_compressed(list_v.at[pl.ds(cnt, 16)], packed, mask=m)
                n = plsc.all_reduce_population_count(m)
                return cnt + n[0]

            return lax.fori_loop(0, 64, grp_body, cnt)

        cnt = lax.fori_loop(0, 32, seg_body, jnp.int32(0))
        ngrp = (cnt + 15) >> 4

        def fire(g):
            s = g & 1
            tail = jnp.logical_and(is31, g == nwin_t - 1)

            @pl.when(tail)
            def _():
                pltpu.async_copy(
                    tailT, win_v.at[s].at[:, pl.ds(0, 128)], wsem)

            @pl.when(jnp.logical_not(tail))
            def _():
                for jh in range(8):
                    pltpu.async_copy(
                        tableT.at[pl.ds(jh * 8, 8),
                                  pl.ds(lo + g * _WINW, _WINW)],
                        win_v.at[s].at[pl.ds(jh * 8, 8), :], wsem)

        def wait_win(g):
            s = g & 1
            tail = jnp.logical_and(is31, g == nwin_t - 1)

            @pl.when(tail)
            def _():
                pltpu.make_async_copy(
                    tailT, win_v.at[s].at[:, pl.ds(0, 128)], wsem).wait()

            @pl.when(jnp.logical_not(tail))
            def _():
                for jh in range(8):
                    pltpu.make_async_copy(
                        tableT.at[pl.ds(jh * 8, 8),
                                  pl.ds(lo + g * _WINW, _WINW)],
                        win_v.at[s].at[pl.ds(jh * 8, 8), :], wsem).wait()

        def wait_scat(s):
            pltpu.make_async_copy(
                stage_v.at[s], out_hbm.at[posb_v.at[s]], osem).wait()

        fire(jnp.int32(0))
        fire(jnp.int32(1))

        def win_body(g, sct):
            s = g & 1
            wait_win(g)
            wbase = g * _WINW

            def scan_body(q, nm):
                packed = list_v[pl.ds(q * 16, 16)]
                rel = packed >> 16
                valid = q * 16 + lax.iota(jnp.int32, 16) < cnt
                m = jnp.logical_and(valid, jnp.logical_and(
                    rel >= wbase, rel < wbase + _WINW))

                @pl.when(jnp.any(m))
                def _():
                    plsc.store_compressed(
                        match_v.at[pl.ds(nm, 16)], packed, mask=m)

                n = plsc.all_reduce_population_count(m)
                return nm + n[0]

            nm = lax.fori_loop(0, ngrp, scan_body, jnp.int32(0))

            def ext_body(e, sct):
                ss = sct & 1

                @pl.when(sct >= 2)
                def _():
                    wait_scat(ss)

                packed = match_v[pl.ds(e * 16, 16)]
                lanei = lax.iota(jnp.int32, 16)
                valid = e * 16 + lanei < nm
                rel_in = (packed >> 16) - wbase
                rel_in = jnp.where(valid, rel_in, 0)
                posr = jnp.where(valid, packed & 0xFFFF, _ROWS + lanei)
                st = stage_v.at[ss]
                for j in range(_D):
                    vals = plsc.load_gather(
                        win_v.at[s], [jnp.full((16,), j, jnp.int32), rel_in])
                    plsc.store_scatter(
                        st, [lanei, jnp.full((16,), j, jnp.int32)], vals)
                posb_v[ss, :] = posr
                pltpu.async_copy(st, out_hbm.at[posb_v.at[ss]], osem)
                return sct + 1

            sct = lax.fori_loop(0, (nm + 15) >> 4, ext_body, sct)

            @pl.when(g + 2 < nwin_t)
            def _():
                fire(g + 2)

            return sct

        sct = lax.fori_loop(0, nwin_t, win_body, jnp.int32(0))

        @pl.when(sct >= 2)
        def _():
            wait_scat(sct & 1)

        @pl.when(sct >= 1)
        def _():
            wait_scat((sct - 1) & 1)

    tail_pad = jnp.pad(table[_TAIL:].T, ((0, 0), (0, 128 - (_NUM_ENT - _TAIL))))
    return k(table.T, tail_pad, idx)


_BB = 2048
_NB = _B // _BB


def _tc_body(h_ref, t_ref, rels_ref, m_ref, g_ref, b_ref, w_ref,
             loss_ref, preds_ref, stats_ref, acc_ref):
    ph = pl.program_id(0)
    b = pl.program_id(1)

    Hb = h_ref[...][:, :_D]
    Tb = t_ref[...][:, :_D]

    @pl.when(jnp.logical_and(ph == 0, b == 0))
    def _init():
        stats_ref[...] = jnp.zeros_like(stats_ref)

    @pl.when(ph == 0)
    def _stats():
        stats_ref[0:1, :] += jnp.sum(Hb, axis=0, keepdims=True)
        stats_ref[1:2, :] += jnp.sum(Hb * Hb, axis=0, keepdims=True)
        stats_ref[2:3, :] += jnp.sum(Tb, axis=0, keepdims=True)
        stats_ref[3:4, :] += jnp.sum(Tb * Tb, axis=0, keepdims=True)

    @pl.when(ph == 1)
    def _decode():
        gamma = g_ref[...]
        beta = b_ref[...]
        mH = stats_ref[0:1, :] / _B
        vH = stats_ref[1:2, :] / _B - mH * mH
        mT = stats_ref[2:3, :] / _B
        vT = stats_ref[3:4, :] / _B - mT * mT
        Hn = (Hb - mH) / jnp.sqrt(vH + _EPS) * gamma + beta
        Tn = (Tb - mT) / jnp.sqrt(vT + _EPS) * gamma + beta

        u0 = jnp.dot(Hn, m_ref[0], preferred_element_type=jnp.float32)
        s0 = jnp.sum(u0 * Tn, axis=1)
        u1 = jnp.dot(Hn, m_ref[1], preferred_element_type=jnp.float32)
        s1 = jnp.sum(u1 * Tn, axis=1)

        logits = [s0 * w_ref[0, j] + s1 * w_ref[1, j] for j in range(_NREL)]
        m = logits[0]
        for j in range(1, _NREL):
            m = jnp.maximum(m, logits[j])
        es = [jnp.exp(lg - m) for lg in logits]
        se = es[0]
        for j in range(1, _NREL):
            se = se + es[j]
        lse = m + jnp.log(se)

        rels = rels_ref[...]
        pick = jnp.zeros_like(s0)
        wsum = jnp.zeros_like(s0)
        for j in range(_NREL):
            pick = pick + jnp.where(rels == j, logits[j], 0.0)
            wsum = wsum + (j + 1.0) * es[j]
        preds_ref[...] = wsum / se

        part = jnp.sum(lse - pick)

        @pl.when(b == 0)
        def _():
            acc_ref[0] = part

        @pl.when(b > 0)
        def _():
            acc_ref[0] += part

        @pl.when(b == _NB - 1)
        def _():
            loss_ref[0, 0] = acc_ref[0] / _B


def _tc_decode(rows, rels, rel_mats, gamma, beta, wscal):
    grid = (2, _NB)
    return pl.pallas_call(
        _tc_body,
        grid=grid,
        in_specs=[
            pl.BlockSpec((_BB, 128), lambda ph, b: (b, 0)),
            pl.BlockSpec((_BB, 128), lambda ph, b: (b + _NB, 0)),
            pl.BlockSpec((_BB,), lambda ph, b: (b,)),
            pl.BlockSpec((2, _D, _D), lambda ph, b: (0, 0, 0)),
            pl.BlockSpec((1, _D), lambda ph, b: (0, 0)),
            pl.BlockSpec((1, _D), lambda ph, b: (0, 0)),
            pl.BlockSpec(memory_space=pltpu.SMEM),
        ],
        out_specs=[
            pl.BlockSpec(memory_space=pltpu.SMEM),
            pl.BlockSpec((_BB,), lambda ph, b: (b,)),
        ],
        out_shape=[
            jax.ShapeDtypeStruct((1, 1), jnp.float32),
            jax.ShapeDtypeStruct((_B,), jnp.float32),
        ],
        scratch_shapes=[
            pltpu.VMEM((8, _D), jnp.float32),
            pltpu.SMEM((1,), jnp.float32),
        ],
    )(rows, rows, rels, rel_mats, gamma, beta, wscal)


def kernel(pos_edges, encoder_weight, bn_gamma, bn_beta, rel_embeds, weight_scalars):
    heads = pos_edges[:, 0]
    rels = pos_edges[:, 1]
    tails = pos_edges[:, 2]
    idx = jnp.concatenate([heads, tails])

    rows = _sc_gather(encoder_weight, idx)

    rel_mats = rel_embeds.reshape(2, _D, _D)
    gamma = bn_gamma.reshape(1, _D)
    beta = bn_beta.reshape(1, _D)

    loss_arr, preds = _tc_decode(rows, rels, rel_mats, gamma, beta, weight_scalars)
    return loss_arr[0, 0], preds.reshape(_B, 1)
